# baseline (speedup 1.0000x reference)
import jax
import jax.numpy as jnp
from jax import lax
from jax.experimental import pallas as pl
from jax.experimental.pallas import tpu as pltpu
from jax.experimental.pallas import tpu_sc as plsc

_NUM_CORES = 2
_NUM_SUBCORES = 16
_LANES = 16
_NUM_WORKERS = _NUM_CORES * _NUM_SUBCORES

_B, _S, _D = 4, 4096, 1024
_ROWS_PER_W = _S // _NUM_WORKERS
_R = 16
_TILE = _R * _D
_NTILES = _ROWS_PER_W // _R


def _sc_body(x_hbm, t_hbm, o_hbm, t_buf, x_buf):
    wid = lax.axis_index("s") * _NUM_CORES + lax.axis_index("c")
    base = wid * (_ROWS_PER_W * _D)

    def add_tile(x_ref, t_ref):
        def body(i, _):
            off = i * _LANES
            x_ref[pl.ds(off, _LANES)] = (
                x_ref[pl.ds(off, _LANES)] + t_ref[pl.ds(off, _LANES)]
            )
            return 0

        lax.fori_loop(0, _TILE // _LANES, body, 0)

    for j in range(_NTILES):
        toff = base + j * _TILE
        pltpu.sync_copy(t_hbm.at[pl.ds(toff, _TILE)], t_buf)
        for b in range(_B):
            xoff = b * (_S * _D) + toff
            pltpu.sync_copy(x_hbm.at[pl.ds(xoff, _TILE)], x_buf)
            add_tile(x_buf, t_buf)
            pltpu.sync_copy(x_buf, o_hbm.at[pl.ds(xoff, _TILE)])


def kernel(inputs, pos_table):
    B, S, D = inputs.shape
    x_flat = inputs.reshape(B * S * D)
    t_flat = pos_table.reshape(S * D)

    mesh = plsc.VectorSubcoreMesh(
        core_axis_name="c", subcore_axis_name="s",
        num_cores=_NUM_CORES, num_subcores=_NUM_SUBCORES,
    )
    out = pl.kernel(
        _sc_body,
        out_type=jax.ShapeDtypeStruct((B * S * D,), jnp.float32),
        mesh=mesh,
        scratch_types=[
            pltpu.VMEM((_TILE,), jnp.float32),
            pltpu.VMEM((_TILE,), jnp.float32),
        ],
    )(x_flat, t_flat)
    return out.reshape(B, S, D)


# speedup vs baseline: 1.8167x; 1.8167x over previous
import jax
import jax.numpy as jnp
from jax import lax
from jax.experimental import pallas as pl
from jax.experimental.pallas import tpu as pltpu
from jax.experimental.pallas import tpu_sc as plsc

_NUM_CORES = 2
_NUM_SUBCORES = 16
_LANES = 16
_NUM_WORKERS = _NUM_CORES * _NUM_SUBCORES

_B, _S, _D = 4, 4096, 1024
_ROWS_PER_W = _S // _NUM_WORKERS
_R = 16
_TILE = _R * _D
_NTILES = _ROWS_PER_W // _R
_NSTEPS = _NTILES * _B
_NBIN = 3
_NBOUT = 2
_NBT = 2


def _add_tile(xi_ref, t_ref, xo_ref):
    @plsc.parallel_loop(0, _TILE, step=_LANES, unroll=8)
    def _(i):
        xo_ref[pl.ds(i, _LANES)] = (
            xi_ref[pl.ds(i, _LANES)] + t_ref[pl.ds(i, _LANES)]
        )


def _sc_body(x_hbm, t_hbm, o_hbm,
             xi0, xi1, xi2, xo0, xo1, tb0, tb1,
             li0, li1, li2, so0, so1, ts0, ts1):
    wid = lax.axis_index("s") * _NUM_CORES + lax.axis_index("c")
    base = wid * (_ROWS_PER_W * _D)

    xin, xout, tbuf = [xi0, xi1, xi2], [xo0, xo1], [tb0, tb1]
    lsem, ssem, tsem = [li0, li1, li2], [so0, so1], [ts0, ts1]

    def t_load(j):
        return pltpu.async_copy(
            t_hbm.at[pl.ds(base + j * _TILE, _TILE)],
            tbuf[j % _NBT], tsem[j % _NBT])

    def x_off(s):
        j, b = s // _B, s % _B
        return b * (_S * _D) + base + j * _TILE

    def x_load(s):
        return pltpu.async_copy(
            x_hbm.at[pl.ds(x_off(s), _TILE)],
            xin[s % _NBIN], lsem[s % _NBIN])

    def x_store(s):
        return pltpu.async_copy(
            xout[s % _NBOUT],
            o_hbm.at[pl.ds(x_off(s), _TILE)], ssem[s % _NBOUT])

    tdesc = {0: t_load(0), 1: t_load(1)}
    xdesc = {s: x_load(s) for s in range(_NBIN)}
    sdesc = {}

    for s in range(_NSTEPS):
        j, b = s // _B, s % _B
        if s - _NBOUT in sdesc:
            sdesc[s - _NBOUT].wait()
        if b == 0:
            tdesc[j].wait()
        xdesc[s].wait()
        _add_tile(xin[s % _NBIN], tbuf[j % _NBT], xout[s % _NBOUT])
        sdesc[s] = x_store(s)
        if s + _NBIN < _NSTEPS:
            xdesc[s + _NBIN] = x_load(s + _NBIN)
        if b == _B - 1 and j + _NBT < _NTILES:
            tdesc[j + _NBT] = t_load(j + _NBT)

    for s in range(_NSTEPS - _NBOUT, _NSTEPS):
        sdesc[s].wait()


def kernel(inputs, pos_table):
    B, S, D = inputs.shape
    x_flat = inputs.reshape(B * S * D)
    t_flat = pos_table.reshape(S * D)

    mesh = plsc.VectorSubcoreMesh(
        core_axis_name="c", subcore_axis_name="s",
        num_cores=_NUM_CORES, num_subcores=_NUM_SUBCORES,
    )
    out = pl.kernel(
        _sc_body,
        out_type=jax.ShapeDtypeStruct((B * S * D,), jnp.float32),
        mesh=mesh,
        scratch_types=(
            [pltpu.VMEM((_TILE,), jnp.float32)] * (_NBIN + _NBOUT + _NBT)
            + [pltpu.SemaphoreType.DMA] * (_NBIN + _NBOUT + _NBT)
        ),
    )(x_flat, t_flat)
    return out.reshape(B, S, D)


# speedup vs baseline: 5.0353x; 2.7717x over previous
import jax
import jax.numpy as jnp
from jax import lax
from jax.experimental import pallas as pl
from jax.experimental.pallas import tpu as pltpu
from jax.experimental.pallas import tpu_sc as plsc

_NUM_CORES = 2
_NUM_SUBCORES = 16
_LANES = 16
_NUM_WORKERS = _NUM_CORES * _NUM_SUBCORES

_B, _S, _D = 4, 4096, 1024
_ROWS_PER_W = _S // _NUM_WORKERS
_R = 16
_NTILES = _ROWS_PER_W // _R
_NSTEPS = _NTILES * _B
_NBIN = 3
_NBOUT = 2
_NBT = 2


def _add_tile(xi_ref, t_ref, xo_ref):
    @plsc.parallel_loop(0, _R * _D, step=_LANES, unroll=8)
    def _(i):
        r = i >> 10
        c = pl.multiple_of(i & (_D - 1), _LANES)
        xo_ref[r, pl.ds(c, _LANES)] = (
            xi_ref[r, pl.ds(c, _LANES)] + t_ref[r, pl.ds(c, _LANES)]
        )


def _sc_body(x_hbm, t_hbm, o_hbm,
             xi0, xi1, xi2, xo0, xo1, tb0, tb1,
             li0, li1, li2, so0, so1, ts0, ts1):
    wid = lax.axis_index("s") * _NUM_CORES + lax.axis_index("c")
    row0 = wid * _ROWS_PER_W

    xin, xout, tbuf = [xi0, xi1, xi2], [xo0, xo1], [tb0, tb1]
    lsem, ssem, tsem = [li0, li1, li2], [so0, so1], [ts0, ts1]

    def t_load(j):
        return pltpu.async_copy(
            t_hbm.at[pl.ds(row0 + j * _R, _R)],
            tbuf[j % _NBT], tsem[j % _NBT])

    def x_load(s):
        j, b = s // _B, s % _B
        return pltpu.async_copy(
            x_hbm.at[b, pl.ds(row0 + j * _R, _R)],
            xin[s % _NBIN], lsem[s % _NBIN])

    def x_store(s):
        j, b = s // _B, s % _B
        return pltpu.async_copy(
            xout[s % _NBOUT],
            o_hbm.at[b, pl.ds(row0 + j * _R, _R)], ssem[s % _NBOUT])

    tdesc = {0: t_load(0), 1: t_load(1)}
    xdesc = {s: x_load(s) for s in range(_NBIN)}
    sdesc = {}

    for s in range(_NSTEPS):
        j, b = s // _B, s % _B
        if s - _NBOUT in sdesc:
            sdesc[s - _NBOUT].wait()
        if b == 0:
            tdesc[j].wait()
        xdesc[s].wait()
        _add_tile(xin[s % _NBIN], tbuf[j % _NBT], xout[s % _NBOUT])
        sdesc[s] = x_store(s)
        if s + _NBIN < _NSTEPS:
            xdesc[s + _NBIN] = x_load(s + _NBIN)
        if b == _B - 1 and j + _NBT < _NTILES:
            tdesc[j + _NBT] = t_load(j + _NBT)

    for s in range(_NSTEPS - _NBOUT, _NSTEPS):
        sdesc[s].wait()


def kernel(inputs, pos_table):
    B, S, D = inputs.shape

    mesh = plsc.VectorSubcoreMesh(
        core_axis_name="c", subcore_axis_name="s",
        num_cores=_NUM_CORES, num_subcores=_NUM_SUBCORES,
    )
    return pl.kernel(
        _sc_body,
        out_type=jax.ShapeDtypeStruct((B, S, D), jnp.float32),
        mesh=mesh,
        compiler_params=pltpu.CompilerParams(use_tc_tiling_on_sc=True),
        scratch_types=(
            [pltpu.VMEM((_R, _D), jnp.float32)] * (_NBIN + _NBOUT + _NBT)
            + [pltpu.SemaphoreType.DMA] * (_NBIN + _NBOUT + _NBT)
        ),
    )(inputs, pos_table)
